# Initial kernel scaffold; baseline (speedup 1.0000x reference)
import jax
import jax.numpy as jnp
from jax.experimental import pallas as pl


def kernel(y_pred, y_true):
    raise NotImplementedError("write your pallas kernel here")


# speedup vs baseline: 18.6785x; 18.6785x over previous
import functools

import jax
import jax.numpy as jnp
from jax import lax
from jax.experimental import pallas as pl
from jax.experimental.pallas import tpu as pltpu
from jax.experimental.pallas import tpu_sc as plsc

NB = 8192
HI = 8.0
SCALE = NB / HI
L = 16
CH = 16384


N_HALF = 131072


def _lovasz_sc_kernel(logits_hbm, labels_hbm, out_hbm,
                      lg_buf, lb_buf, c_h, p_h, s_h,
                      pc_h, pp_h, ps_h, gbuf, obuf, shared, gshare):
    c_ax = lax.axis_index("c")
    s_ax = lax.axis_index("s")
    img = c_ax * 8 + s_ax // 2
    half = s_ax % 2

    zeros = jnp.zeros((L,), jnp.float32)
    ones = jnp.full((L,), 1.0, jnp.float32)

    def zero_body(i, _):
        sl = pl.ds(i * L, L)
        c_h[sl] = zeros
        p_h[sl] = zeros
        s_h[sl] = zeros
        return 0
    lax.fori_loop(0, NB // L, zero_body, 0)

    n_chunks = N_HALF // CH
    base = (img * 2 + half) * N_HALF

    def chunk_body(ci, gacc):
        pltpu.sync_copy(logits_hbm.at[pl.ds(base + ci * CH, CH)], lg_buf)
        pltpu.sync_copy(labels_hbm.at[pl.ds(base + ci * CH, CH)], lb_buf)

        def vec_body(v, g):
            sl = pl.ds(v * L, L)
            x = lg_buf[sl]
            t = lb_buf[sl]
            e = 1.0 - x * (2.0 * t - 1.0)
            m = e > 0.0
            k = jnp.clip((e * SCALE).astype(jnp.int32), 0, NB - 1)
            idx = (NB - 1) - k
            plsc.addupdate_scatter(c_h, [idx], ones, mask=m)
            plsc.addupdate_scatter(p_h, [idx], t, mask=m)
            plsc.addupdate_scatter(s_h, [idx], e, mask=m)
            return g + t
        return lax.fori_loop(0, CH // L, vec_body, gacc)

    gacc = lax.fori_loop(0, n_chunks, chunk_body, zeros)

    gbuf[pl.ds(0, L)] = gacc
    srow = s_ax * (3 * NB)
    pltpu.sync_copy(c_h, shared.at[pl.ds(srow, NB)])
    pltpu.sync_copy(p_h, shared.at[pl.ds(srow + NB, NB)])
    pltpu.sync_copy(s_h, shared.at[pl.ds(srow + 2 * NB, NB)])
    pltpu.sync_copy(gbuf, gshare.at[pl.ds(s_ax * L, L)])
    plsc.subcore_barrier()

    @pl.when(half == 0)
    def _():
        prow = (s_ax + 1) * (3 * NB)
        pltpu.sync_copy(shared.at[pl.ds(prow, NB)], pc_h)
        pltpu.sync_copy(shared.at[pl.ds(prow + NB, NB)], pp_h)
        pltpu.sync_copy(shared.at[pl.ds(prow + 2 * NB, NB)], ps_h)
        pltpu.sync_copy(gshare.at[pl.ds((s_ax + 1) * L, L)], gbuf)
        G = jnp.sum(gacc) + jnp.sum(gbuf[pl.ds(0, L)])

        def scan_body(i, carry):
            i_run, p_run, acc = carry
            sl = pl.ds(i * L, L)
            c = c_h[sl] + pc_h[sl]
            p = p_h[sl] + pp_h[sl]
            s = s_h[sl] + ps_h[sl]
            I = plsc.cumsum(c) + i_run
            P = plsc.cumsum(p) + p_run
            I0 = I - c
            P0 = P - p
            J1 = jnp.where(I > 0.0,
                           1.0 - (G - P) / jnp.maximum(G + I - P, 1.0), 0.0)
            J0 = jnp.where(I0 > 0.0,
                           1.0 - (G - P0) / jnp.maximum(G + I0 - P0, 1.0), 0.0)
            acc = acc + (s / jnp.maximum(c, 1.0)) * (J1 - J0)
            return (i_run + jnp.sum(c), p_run + jnp.sum(p), acc)

        init = (jnp.float32(0.0), jnp.float32(0.0), zeros)
        _, _, acc = lax.fori_loop(0, NB // L, scan_body, init)
        loss = jnp.sum(acc)
        obuf[pl.ds(0, L)] = jnp.broadcast_to(loss, (L,))
        pltpu.sync_copy(obuf, out_hbm.at[pl.ds(img * L, L)])


@jax.jit
def _lovasz_sc(logits, labels):
    mesh = plsc.VectorSubcoreMesh(core_axis_name="c", subcore_axis_name="s")
    f = functools.partial(
        pl.kernel,
        out_type=jax.ShapeDtypeStruct((16 * L,), jnp.float32),
        mesh=mesh,
        compiler_params=pltpu.CompilerParams(needs_layout_passes=False),
        scratch_types=[
            pltpu.VMEM((CH,), jnp.float32),
            pltpu.VMEM((CH,), jnp.float32),
            pltpu.VMEM((NB,), jnp.float32),
            pltpu.VMEM((NB,), jnp.float32),
            pltpu.VMEM((NB,), jnp.float32),
            pltpu.VMEM((NB,), jnp.float32),
            pltpu.VMEM((NB,), jnp.float32),
            pltpu.VMEM((NB,), jnp.float32),
            pltpu.VMEM((L,), jnp.float32),
            pltpu.VMEM((L,), jnp.float32),
            pltpu.VMEM_SHARED((16 * 3 * NB,), jnp.float32),
            pltpu.VMEM_SHARED((16 * L,), jnp.float32),
        ],
    )(_lovasz_sc_kernel)
    return f(logits, labels)


def kernel(y_pred, y_true):
    logits = y_pred.astype(jnp.float32).reshape(-1)
    labels = y_true.astype(jnp.float32).reshape(-1)
    out = _lovasz_sc(logits, labels)
    return jnp.mean(out.reshape(16, L)[:, 0])


# speedup vs baseline: 20.2116x; 1.0821x over previous
import functools

import jax
import jax.numpy as jnp
from jax import lax
from jax.experimental import pallas as pl
from jax.experimental.pallas import tpu as pltpu
from jax.experimental.pallas import tpu_sc as plsc

NB = 4096
NB2 = 2 * NB
HI = 8.0
SCALE = NB / HI
L = 16
CH = 16384
N_HALF = 131072
N_CHUNKS = N_HALF // CH
UNROLL = 8


def _lovasz_sc_kernel(logits_hbm, labels_hbm, out_hbm,
                      lg0, lg1, lb0, lb1, h2, se2, ph2, pse2,
                      gbuf, obuf, shared, gshare, sem0, sem1):
    c_ax = lax.axis_index("c")
    s_ax = lax.axis_index("s")
    img = c_ax * 8 + s_ax // 2
    half = s_ax % 2

    zeros = jnp.zeros((L,), jnp.float32)
    ones = jnp.full((L,), 1.0, jnp.float32)

    def zero_body(i, _):
        sl = pl.ds(i * L, L)
        h2[sl] = zeros
        se2[sl] = zeros
        return 0
    lax.fori_loop(0, NB2 // L, zero_body, 0)

    base = (img * 2 + half) * N_HALF
    lg = (lg0, lg1)
    lb = (lb0, lb1)
    sems = (sem0, sem1)

    def issue(ci, slot):
        off = base + ci * CH
        pltpu.async_copy(logits_hbm.at[pl.ds(off, CH)], lg[slot], sems[slot])
        pltpu.async_copy(labels_hbm.at[pl.ds(off, CH)], lb[slot], sems[slot])

    def wait(slot):
        pltpu.make_async_copy(
            logits_hbm.at[pl.ds(0, CH)], lg[slot], sems[slot]).wait()
        pltpu.make_async_copy(
            labels_hbm.at[pl.ds(0, CH)], lb[slot], sems[slot]).wait()

    def chunk_compute(slot, gacc):
        lgb, lbb = lg[slot], lb[slot]

        def vec_body(v, g):
            for u in range(UNROLL):
                sl = pl.ds((v * UNROLL + u) * L, L)
                x = lgb[sl]
                t = lbb[sl]
                e = 1.0 - x * (2.0 * t - 1.0)
                m = e > 0.0
                k = jnp.minimum((e * SCALE).astype(jnp.int32), NB - 1)
                idx = (NB2 - 2) - 2 * k + t.astype(jnp.int32)
                plsc.addupdate_scatter(h2, [idx], ones, mask=m)
                plsc.addupdate_scatter(se2, [idx], e, mask=m)
                g = g + t
            return g
        return lax.fori_loop(0, CH // (L * UNROLL), vec_body, gacc)

    issue(0, 0)
    gacc = zeros
    for ci in range(N_CHUNKS):
        slot = ci % 2
        wait(slot)
        if ci + 1 < N_CHUNKS:
            issue(ci + 1, 1 - slot)
        gacc = chunk_compute(slot, gacc)

    gbuf[pl.ds(0, L)] = gacc
    srow = s_ax * (2 * NB2)
    pltpu.sync_copy(h2, shared.at[pl.ds(srow, NB2)])
    pltpu.sync_copy(se2, shared.at[pl.ds(srow + NB2, NB2)])
    pltpu.sync_copy(gbuf, gshare.at[pl.ds(s_ax * L, L)])
    plsc.subcore_barrier()

    @pl.when(half == 0)
    def _():
        prow = (s_ax + 1) * (2 * NB2)
        pltpu.sync_copy(shared.at[pl.ds(prow, NB2)], ph2)
        pltpu.sync_copy(shared.at[pl.ds(prow + NB2, NB2)], pse2)
        pltpu.sync_copy(gshare.at[pl.ds((s_ax + 1) * L, L)], gbuf)
        G = jnp.sum(gacc) + jnp.sum(gbuf[pl.ds(0, L)])
        par = (lax.iota(jnp.int32, L) % 2).astype(jnp.float32)

        def scan_body(i, carry):
            i_run, p_run, acc = carry
            sl = pl.ds(i * L, L)
            c = h2[sl] + ph2[sl]
            s = se2[sl] + pse2[sl]
            p = c * par
            I = plsc.cumsum(c) + i_run
            P = plsc.cumsum(p) + p_run
            I0 = I - c
            P0 = P - p
            J1 = jnp.where(I > 0.0,
                           1.0 - (G - P) / jnp.maximum(G + I - P, 1.0), 0.0)
            J0 = jnp.where(I0 > 0.0,
                           1.0 - (G - P0) / jnp.maximum(G + I0 - P0, 1.0), 0.0)
            acc = acc + (s / jnp.maximum(c, 1.0)) * (J1 - J0)
            return (i_run + jnp.sum(c), p_run + jnp.sum(p), acc)

        init = (jnp.float32(0.0), jnp.float32(0.0), zeros)
        _, _, acc = lax.fori_loop(0, NB2 // L, scan_body, init)
        loss = jnp.sum(acc)
        obuf[pl.ds(0, L)] = jnp.broadcast_to(loss, (L,))
        pltpu.sync_copy(obuf, out_hbm.at[pl.ds(img * L, L)])


@jax.jit
def _lovasz_sc(logits, labels):
    mesh = plsc.VectorSubcoreMesh(core_axis_name="c", subcore_axis_name="s")
    f = functools.partial(
        pl.kernel,
        out_type=jax.ShapeDtypeStruct((16 * L,), jnp.float32),
        mesh=mesh,
        compiler_params=pltpu.CompilerParams(needs_layout_passes=False),
        scratch_types=[
            pltpu.VMEM((CH,), jnp.float32),
            pltpu.VMEM((CH,), jnp.float32),
            pltpu.VMEM((CH,), jnp.float32),
            pltpu.VMEM((CH,), jnp.float32),
            pltpu.VMEM((NB2,), jnp.float32),
            pltpu.VMEM((NB2,), jnp.float32),
            pltpu.VMEM((NB2,), jnp.float32),
            pltpu.VMEM((NB2,), jnp.float32),
            pltpu.VMEM((L,), jnp.float32),
            pltpu.VMEM((L,), jnp.float32),
            pltpu.VMEM_SHARED((16 * 2 * NB2,), jnp.float32),
            pltpu.VMEM_SHARED((16 * L,), jnp.float32),
            pltpu.SemaphoreType.DMA,
            pltpu.SemaphoreType.DMA,
        ],
    )(_lovasz_sc_kernel)
    return f(logits, labels)


def kernel(y_pred, y_true):
    logits = y_pred.astype(jnp.float32).reshape(-1)
    labels = y_true.astype(jnp.float32).reshape(-1)
    out = _lovasz_sc(logits, labels)
    return jnp.mean(out.reshape(16, L)[:, 0])


# speedup vs baseline: 44.9270x; 2.2228x over previous
import functools

import jax
import jax.numpy as jnp
from jax import lax
from jax.experimental import pallas as pl
from jax.experimental.pallas import tpu as pltpu
from jax.experimental.pallas import tpu_sc as plsc

NB = 4096
NB2 = 2 * NB
HI = 8.0
SCALE = NB / HI
L = 16
CH = 16384
N_HALF = 131072
N_CHUNKS = N_HALF // CH
UNROLL = 8


def _lovasz_sc_kernel(logits_hbm, labels_hbm, out_hbm,
                      lg0, lg1, lb0, lb1, h2, se2, ph2, pse2,
                      gbuf, obuf, shared, gshare, sem0, sem1):
    c_ax = lax.axis_index("c")
    s_ax = lax.axis_index("s")
    img = c_ax * 8 + s_ax // 2
    half = s_ax % 2

    zeros = jnp.zeros((L,), jnp.float32)
    ones = jnp.full((L,), 1.0, jnp.float32)

    def zero_body(i, _):
        sl = pl.ds(i * L, L)
        h2[sl] = zeros
        se2[sl] = zeros
        return 0
    lax.fori_loop(0, NB2 // L, zero_body, 0)

    base = (img * 2 + half) * N_HALF
    lg = (lg0, lg1)
    lb = (lb0, lb1)
    sems = (sem0, sem1)

    def issue(ci, slot):
        off = base + ci * CH
        pltpu.async_copy(logits_hbm.at[pl.ds(off, CH)], lg[slot], sems[slot])
        pltpu.async_copy(labels_hbm.at[pl.ds(off, CH)], lb[slot], sems[slot])

    def wait(slot):
        pltpu.make_async_copy(
            logits_hbm.at[pl.ds(0, CH)], lg[slot], sems[slot]).wait()
        pltpu.make_async_copy(
            labels_hbm.at[pl.ds(0, CH)], lb[slot], sems[slot]).wait()

    def chunk_compute(slot, gacc):
        lgb, lbb = lg[slot], lb[slot]

        def vec_body(v, g):
            sl = pl.ds(v * L, L)
            x = lgb[sl]
            t = lbb[sl]
            k = (x * SCALE + SCALE) - (x * (2.0 * SCALE)) * t
            m = k > 0.0
            e = k * (1.0 / SCALE)
            ki = jnp.minimum(k, float(NB - 1)).astype(jnp.int32)
            idx = (NB2 - 2) - 2 * ki + t.astype(jnp.int32)
            plsc.addupdate_scatter(h2, [idx], ones, mask=m)
            plsc.addupdate_scatter(se2, [idx], e, mask=m)
            return g + t
        return plsc.parallel_loop(
            0, CH // L, 1, unroll=UNROLL, carry=gacc)(vec_body)

    issue(0, 0)
    gacc = zeros
    for ci in range(N_CHUNKS):
        slot = ci % 2
        wait(slot)
        if ci + 1 < N_CHUNKS:
            issue(ci + 1, 1 - slot)
        gacc = chunk_compute(slot, gacc)

    gbuf[pl.ds(0, L)] = gacc
    srow = s_ax * (2 * NB2)
    pltpu.sync_copy(h2, shared.at[pl.ds(srow, NB2)])
    pltpu.sync_copy(se2, shared.at[pl.ds(srow + NB2, NB2)])
    pltpu.sync_copy(gbuf, gshare.at[pl.ds(s_ax * L, L)])
    plsc.subcore_barrier()

    @pl.when(half == 0)
    def _():
        prow = (s_ax + 1) * (2 * NB2)
        pltpu.sync_copy(shared.at[pl.ds(prow, NB2)], ph2)
        pltpu.sync_copy(shared.at[pl.ds(prow + NB2, NB2)], pse2)
        pltpu.sync_copy(gshare.at[pl.ds((s_ax + 1) * L, L)], gbuf)
        G = jnp.sum(gacc) + jnp.sum(gbuf[pl.ds(0, L)])
        par = (lax.iota(jnp.int32, L) % 2).astype(jnp.float32)

        def scan_body(i, carry):
            i_run, p_run, acc = carry
            sl = pl.ds(i * L, L)
            c = h2[sl] + ph2[sl]
            s = se2[sl] + pse2[sl]
            p = c * par
            I = plsc.cumsum(c) + i_run
            P = plsc.cumsum(p) + p_run
            I0 = I - c
            P0 = P - p
            J1 = jnp.where(I > 0.0,
                           1.0 - (G - P) / jnp.maximum(G + I - P, 1.0), 0.0)
            J0 = jnp.where(I0 > 0.0,
                           1.0 - (G - P0) / jnp.maximum(G + I0 - P0, 1.0), 0.0)
            acc = acc + (s / jnp.maximum(c, 1.0)) * (J1 - J0)
            return (i_run + jnp.sum(c), p_run + jnp.sum(p), acc)

        init = (jnp.float32(0.0), jnp.float32(0.0), zeros)
        _, _, acc = lax.fori_loop(0, NB2 // L, scan_body, init)
        loss = jnp.sum(acc)
        obuf[pl.ds(0, L)] = jnp.broadcast_to(loss, (L,))
        pltpu.sync_copy(obuf, out_hbm.at[pl.ds(img * L, L)])


@jax.jit
def _lovasz_sc(logits, labels):
    mesh = plsc.VectorSubcoreMesh(core_axis_name="c", subcore_axis_name="s")
    f = functools.partial(
        pl.kernel,
        out_type=jax.ShapeDtypeStruct((16 * L,), jnp.float32),
        mesh=mesh,
        compiler_params=pltpu.CompilerParams(needs_layout_passes=False),
        scratch_types=[
            pltpu.VMEM((CH,), jnp.float32),
            pltpu.VMEM((CH,), jnp.float32),
            pltpu.VMEM((CH,), jnp.float32),
            pltpu.VMEM((CH,), jnp.float32),
            pltpu.VMEM((NB2,), jnp.float32),
            pltpu.VMEM((NB2,), jnp.float32),
            pltpu.VMEM((NB2,), jnp.float32),
            pltpu.VMEM((NB2,), jnp.float32),
            pltpu.VMEM((L,), jnp.float32),
            pltpu.VMEM((L,), jnp.float32),
            pltpu.VMEM_SHARED((16 * 2 * NB2,), jnp.float32),
            pltpu.VMEM_SHARED((16 * L,), jnp.float32),
            pltpu.SemaphoreType.DMA,
            pltpu.SemaphoreType.DMA,
        ],
    )(_lovasz_sc_kernel)
    return f(logits, labels)


def kernel(y_pred, y_true):
    logits = y_pred.astype(jnp.float32).reshape(-1)
    labels = y_true.astype(jnp.float32).reshape(-1)
    out = _lovasz_sc(logits, labels)
    return jnp.mean(out.reshape(16, L)[:, 0])
